# baseline (speedup 1.0000x reference)
import functools

import jax
import jax.numpy as jnp
from jax import lax
from jax.experimental import pallas as pl
from jax.experimental.pallas import tpu as pltpu
from jax.experimental.pallas import tpu_sc as plsc

B, S, E = 16, 512, 256
FILT = 256
N_BINS = 256
MAXL = 2048
NC, NS = 2, 16
NW = NC * NS
FPW = (B * MAXL) // NW
CH = 128


def _a_body(sl_ref, x_ref, d_ref, pt_ref, et_ref, pemb_ref, eemb_ref,
            blo_p_ref, bhi_p_ref, blo_e_ref, bhi_e_ref,
            x2_ref, gidx_ref, mlen_ref, mask_ref):
    b = pl.program_id(0)

    @pl.when(b == B)
    def _zero_block():
        x2_ref[...] = jnp.zeros((1, S, E), jnp.float32)

    @pl.when(b < B)
    def _main():
        r = lax.rem(b, 8)
        x = x_ref[0]
        sl = sl_ref[b]

        pt = pt_ref[pl.ds(r, 1), :]
        ohT_p = ((pt >= blo_p_ref[...]).astype(jnp.float32)
                 - (pt >= bhi_p_ref[...]).astype(jnp.float32))
        et = et_ref[pl.ds(r, 1), :]
        ohT_e = ((et >= blo_e_ref[...]).astype(jnp.float32)
                 - (et >= bhi_e_ref[...]).astype(jnp.float32))

        def dotT(ohT, emb):
            return lax.dot_general(ohT, emb, (((0,), (0,)), ((), ())),
                                   preferred_element_type=jnp.float32)

        x2_ref[0] = x + dotT(ohT_p, pemb_ref[...]) + dotT(ohT_e, eemb_ref[...])

        drow = d_ref[pl.ds(r, 1), :].astype(jnp.float32)
        tokr = lax.broadcasted_iota(jnp.int32, (1, S), 1)
        dmask = jnp.where(tokr >= sl, 0.0, drow)
        ii = lax.broadcasted_iota(jnp.int32, (S, S), 0)
        jj = lax.broadcasted_iota(jnp.int32, (S, S), 1)
        ltri = (jj <= ii).astype(jnp.float32)
        cum_col = lax.dot_general(ltri, dmask, (((1,), (1,)), ((), ())),
                                  preferred_element_type=jnp.float32)

        total = jnp.sum(dmask).astype(jnp.int32)
        mlen = jnp.minimum(total, MAXL)
        mlen_ref[0] = jnp.full((1, 128), mlen, jnp.int32)

        frames = lax.broadcasted_iota(jnp.int32, (1, MAXL), 1)
        gefT = (cum_col <= frames.astype(jnp.float32)).astype(jnp.float32)
        ones = jnp.full((1, S), 1, jnp.float32)
        idxf = lax.dot_general(ones, gefT, (((1,), (0,)), ((), ())),
                               preferred_element_type=jnp.float32)
        idx = jnp.clip(idxf.astype(jnp.int32), 0, S - 1)
        zidx = B * S + (frames & (S - 1))
        gidx_ref[pl.ds(r, 1), :] = jnp.where(frames < mlen, idx + b * S, zidx)
        mask_ref[pl.ds(r, 1), :] = (frames >= mlen).astype(jnp.int32)


def _a_call(src_lens, x, dur, pt, et, pemb, eemb,
            blo_p, bhi_p, blo_e, bhi_e):
    out_shape = (
        jax.ShapeDtypeStruct((B + 1, S, E), jnp.float32),
        jax.ShapeDtypeStruct((B, MAXL), jnp.int32),
        jax.ShapeDtypeStruct((B, 1, 128), jnp.int32),
        jax.ShapeDtypeStruct((B, MAXL), jnp.int32),
    )

    def full(shape):
        return pl.BlockSpec(shape, lambda b, n=len(shape): (0,) * n)

    def row8(n):
        return pl.BlockSpec((8, n), lambda b: (jnp.minimum(b, B - 1) // 8, 0))

    return pl.pallas_call(
        _a_body,
        grid=(B + 1,),
        in_specs=[
            pl.BlockSpec(memory_space=pltpu.SMEM),
            pl.BlockSpec((1, S, E),
                         lambda b: (jnp.minimum(b, B - 1), 0, 0)),
            row8(S),
            row8(S),
            row8(S),
            full((N_BINS, E)), full((N_BINS, E)),
            full((N_BINS, 1)), full((N_BINS, 1)),
            full((N_BINS, 1)), full((N_BINS, 1)),
        ],
        out_specs=[
            pl.BlockSpec((1, S, E), lambda b: (b, 0, 0)),
            row8(MAXL),
            pl.BlockSpec((1, 1, 128),
                         lambda b: (jnp.minimum(b, B - 1), 0, 0)),
            row8(MAXL),
        ],
        out_shape=out_shape,
    )(src_lens, x, dur, pt, et, pemb, eemb, blo_p, bhi_p, blo_e, bhi_e)


def _b_body(sl_ref, blp_ref, gwsum_ref, x_ref, w1_ref, b1_ref,
            g1_ref, be1_ref, w2_ref, b2_ref, wlp_ref,
            logd_ref, pp_ref, ep_ref):
    b = pl.program_id(0)
    r = lax.rem(b, 8)
    x = x_ref[0]
    sl = sl_ref[b]
    padr = lax.broadcasted_iota(jnp.int32, (1, S), 1) >= sl

    zrow = jnp.zeros((1, E), jnp.float32)
    xcat = jnp.concatenate(
        [jnp.concatenate([zrow, x[:-1]], axis=0), x,
         jnp.concatenate([x[1:], zrow], axis=0)], axis=1)
    xcat_b = xcat.astype(jnp.bfloat16)

    def mmf32(a, w):
        return lax.dot_general(a, w, (((1,), (0,)), ((), ())),
                               preferred_element_type=jnp.float32)

    def rowdot(a, h):
        return lax.dot_general(a, h, (((1,), (1,)), ((), ())),
                               preferred_element_type=jnp.float32)

    ones_row = jnp.full((1, FILT), 1.0 / FILT, jnp.float32)

    def predictor(i, out_ref):
        h = jnp.maximum(mmf32(xcat_b, w1_ref[i]) + b1_ref[i][None, :], 0.0)
        m = jnp.mean(h, axis=-1, keepdims=True)
        q = jnp.mean(h * h, axis=-1, keepdims=True)
        s = lax.rsqrt(q - m * m + 1e-5)
        v = ((h - m) * s * g1_ref[i][None, :]
             + be1_ref[i][None, :]).astype(jnp.bfloat16)
        zr = jnp.zeros((1, FILT), jnp.bfloat16)
        vcat = jnp.concatenate(
            [jnp.concatenate([zr, v[:-1]], axis=0), v,
             jnp.concatenate([v[1:], zr], axis=0)], axis=1)
        h2 = jnp.maximum(mmf32(vcat, w2_ref[i]) + b2_ref[i][None, :], 0.0)
        m2 = rowdot(ones_row, h2)
        q2 = rowdot(ones_row, h2 * h2)
        s2 = lax.rsqrt(q2 - m2 * m2 + 1e-5)
        u = rowdot(wlp_ref[i], h2)
        o = s2 * (u - m2 * gwsum_ref[i, 0]) + blp_ref[i, 0]
        out_ref[pl.ds(r, 1), :] = jnp.where(padr, 0.0, o)

    predictor(0, logd_ref)
    predictor(1, pp_ref)
    predictor(2, ep_ref)


def _b_call(src_lens, blp, gwsum, x, W1r, b1s, g1s, be1s, W2r, b2s, wlp):
    out_shape = (
        jax.ShapeDtypeStruct((B, S), jnp.float32),
        jax.ShapeDtypeStruct((B, S), jnp.float32),
        jax.ShapeDtypeStruct((B, S), jnp.float32),
    )

    def full(shape):
        return pl.BlockSpec(shape, lambda b, n=len(shape): (0,) * n)

    def row8(n):
        return pl.BlockSpec((8, n), lambda b: (b // 8, 0))

    return pl.pallas_call(
        _b_body,
        grid=(B,),
        in_specs=[
            pl.BlockSpec(memory_space=pltpu.SMEM),
            pl.BlockSpec(memory_space=pltpu.SMEM),
            pl.BlockSpec(memory_space=pltpu.SMEM),
            pl.BlockSpec((1, S, E), lambda b: (b, 0, 0)),
            full((3, 3 * E, FILT)),
            full((3, FILT)),
            full((3, FILT)), full((3, FILT)),
            full((3, 3 * FILT, FILT)),
            full((3, FILT)),
            full((3, 1, FILT)),
        ],
        out_specs=[row8(S), row8(S), row8(S)],
        out_shape=out_shape,
    )(src_lens, blp, gwsum, x, W1r, b1s, g1s, be1s, W2r, b2s, wlp)


def _sc_gather(x2f, gidxf):
    mesh = plsc.VectorSubcoreMesh(core_axis_name="c", subcore_axis_name="s")

    @functools.partial(
        pl.kernel,
        mesh=mesh,
        out_type=jax.ShapeDtypeStruct((B * MAXL, E), jnp.float32),
        scratch_types=[
            pltpu.VMEM((FPW,), jnp.int32),
            pltpu.VMEM((CH, E), jnp.float32),
            pltpu.VMEM((CH, E), jnp.float32),
            pltpu.SemaphoreType.DMA,
            pltpu.SemaphoreType.DMA,
        ],
    )
    def k(x2_hbm, gidx_hbm, out_hbm, idx_v, buf0, buf1, sem0, sem1):
        cid = lax.axis_index("c")
        sid = lax.axis_index("s")
        wid = sid * NC + cid
        base = wid * FPW
        pltpu.sync_copy(gidx_hbm.at[pl.ds(base, FPW)], idx_v)
        bufs = (buf0, buf1)
        sems = (sem0, sem1)
        nch = FPW // CH
        cps = [None] * nch
        cps[0] = pltpu.async_copy(x2_hbm.at[idx_v.at[pl.ds(0, CH)]],
                                  bufs[0], sems[0])
        for ci in range(nch):
            if ci + 1 < nch:
                cps[ci + 1] = pltpu.async_copy(
                    x2_hbm.at[idx_v.at[pl.ds((ci + 1) * CH, CH)]],
                    bufs[(ci + 1) % 2], sems[(ci + 1) % 2])
            cps[ci].wait()
            pltpu.sync_copy(bufs[ci % 2],
                            out_hbm.at[pl.ds(base + ci * CH, CH)])

    return k(x2f, gidxf)


def kernel(x, src_lens, duration_target, pitch_target, energy_target,
           max_len, W1s, b1s, g1s, be1s, W2s, b2s, g2s, be2s, Wls, bls,
           pitch_emb, energy_emb, pitch_bins, energy_bins):
    ninf = jnp.full((1,), -jnp.inf, jnp.float32)
    pinf = jnp.full((1,), jnp.inf, jnp.float32)
    blo_p = jnp.concatenate([ninf, pitch_bins]).reshape(N_BINS, 1)
    bhi_p = jnp.concatenate([pitch_bins, pinf]).reshape(N_BINS, 1)
    blo_e = jnp.concatenate([ninf, energy_bins]).reshape(N_BINS, 1)
    bhi_e = jnp.concatenate([energy_bins, pinf]).reshape(N_BINS, 1)

    x2p, gidx2, mlen3, mask2 = _a_call(
        src_lens, x, duration_target, pitch_target, energy_target,
        pitch_emb, energy_emb, blo_p, bhi_p, blo_e, bhi_e)

    mel = _sc_gather(x2p.reshape((B + 1) * S, E), gidx2.reshape(B * MAXL))

    wlp = (g2s * Wls[:, :, 0]).reshape(3, 1, FILT)
    blp = bls + jnp.sum(be2s * Wls[:, :, 0], axis=1, keepdims=True)
    gwsum = jnp.sum(wlp[:, 0, :], axis=1, keepdims=True)

    logd, pp, ep = _b_call(
        src_lens, blp, gwsum, x,
        W1s.reshape(3, 3 * E, FILT).astype(jnp.bfloat16), b1s, g1s, be1s,
        W2s.reshape(3, 3 * FILT, FILT).astype(jnp.bfloat16), b2s, wlp)

    return (mel.reshape(B, MAXL, E), logd, pp, ep,
            mlen3[:, 0, 0], mask2.astype(bool))


# speedup vs baseline: 1.0850x; 1.0850x over previous
import functools

import jax
import jax.numpy as jnp
from jax import lax
from jax.experimental import pallas as pl
from jax.experimental.pallas import tpu as pltpu
from jax.experimental.pallas import tpu_sc as plsc

B, S, E = 16, 512, 256
FILT = 256
N_BINS = 256
MAXL = 2048
NC, NS = 2, 16
NW = NC * NS
FPW = (B * MAXL) // NW
CH = 128


def _a_body(sl_ref, x_ref, d_ref, pt_ref, et_ref, pemb_ref, eemb_ref,
            blo_p_ref, bhi_p_ref, blo_e_ref, bhi_e_ref,
            x2_ref, gidx_ref, mlen_ref, mask_ref):
    b = pl.program_id(0)

    @pl.when(b == B)
    def _zero_block():
        x2_ref[...] = jnp.zeros((1, S, E), jnp.float32)

    @pl.when(b < B)
    def _main():
        r = lax.rem(b, 8)
        x = x_ref[0]
        sl = sl_ref[b]

        pt = pt_ref[pl.ds(r, 1), :]
        ohT_p = ((pt >= blo_p_ref[...]).astype(jnp.float32)
                 - (pt >= bhi_p_ref[...]).astype(jnp.float32))
        et = et_ref[pl.ds(r, 1), :]
        ohT_e = ((et >= blo_e_ref[...]).astype(jnp.float32)
                 - (et >= bhi_e_ref[...]).astype(jnp.float32))

        def dotT(ohT, emb):
            return lax.dot_general(ohT, emb, (((0,), (0,)), ((), ())),
                                   preferred_element_type=jnp.float32)

        x2_ref[0] = x + dotT(ohT_p, pemb_ref[...]) + dotT(ohT_e, eemb_ref[...])

        drow = d_ref[pl.ds(r, 1), :].astype(jnp.float32)
        tokr = lax.broadcasted_iota(jnp.int32, (1, S), 1)
        dmask = jnp.where(tokr >= sl, 0.0, drow)
        ii = lax.broadcasted_iota(jnp.int32, (S, S), 0)
        jj = lax.broadcasted_iota(jnp.int32, (S, S), 1)
        ltri = (jj <= ii).astype(jnp.float32)
        cum_col = lax.dot_general(ltri, dmask, (((1,), (1,)), ((), ())),
                                  preferred_element_type=jnp.float32)

        total = jnp.sum(dmask).astype(jnp.int32)
        mlen = jnp.minimum(total, MAXL)
        mlen_ref[0] = jnp.full((1, 128), mlen, jnp.int32)

        frames = lax.broadcasted_iota(jnp.int32, (1, MAXL), 1)
        gefT = (cum_col <= frames.astype(jnp.float32)).astype(jnp.float32)
        ones = jnp.full((1, S), 1, jnp.float32)
        idxf = lax.dot_general(ones, gefT, (((1,), (0,)), ((), ())),
                               preferred_element_type=jnp.float32)
        idx = jnp.clip(idxf.astype(jnp.int32), 0, S - 1)
        zidx = B * S + (frames & (S - 1))
        gidx_ref[pl.ds(r, 1), :] = jnp.where(frames < mlen, idx + b * S, zidx)
        mask_ref[pl.ds(r, 1), :] = (frames >= mlen).astype(jnp.int32)


def _a_call(src_lens, x, dur, pt, et, pemb, eemb,
            blo_p, bhi_p, blo_e, bhi_e):
    out_shape = (
        jax.ShapeDtypeStruct((B + 1, S, E), jnp.float32),
        jax.ShapeDtypeStruct((B, MAXL), jnp.int32),
        jax.ShapeDtypeStruct((B, 1, 128), jnp.int32),
        jax.ShapeDtypeStruct((B, MAXL), jnp.int32),
    )

    def full(shape):
        return pl.BlockSpec(shape, lambda b, n=len(shape): (0,) * n)

    def row8(n):
        return pl.BlockSpec((8, n), lambda b: (jnp.minimum(b, B - 1) // 8, 0))

    return pl.pallas_call(
        _a_body,
        grid=(B + 1,),
        in_specs=[
            pl.BlockSpec(memory_space=pltpu.SMEM),
            pl.BlockSpec((1, S, E),
                         lambda b: (jnp.minimum(b, B - 1), 0, 0)),
            row8(S),
            row8(S),
            row8(S),
            full((N_BINS, E)), full((N_BINS, E)),
            full((N_BINS, 1)), full((N_BINS, 1)),
            full((N_BINS, 1)), full((N_BINS, 1)),
        ],
        out_specs=[
            pl.BlockSpec((1, S, E), lambda b: (b, 0, 0)),
            row8(MAXL),
            pl.BlockSpec((1, 1, 128),
                         lambda b: (jnp.minimum(b, B - 1), 0, 0)),
            row8(MAXL),
        ],
        out_shape=out_shape,
    )(src_lens, x, dur, pt, et, pemb, eemb, blo_p, bhi_p, blo_e, bhi_e)


def _b_body(sl_ref, blp_ref, gwsum_ref, x_ref, w1_ref, b1_ref, beg1_ref,
            w2_ref, b2_ref, wlp_ref,
            logd_ref, pp_ref, ep_ref):
    b = pl.program_id(0)
    r = lax.rem(b, 8)
    x = x_ref[0]
    sl = sl_ref[b]
    padr = lax.broadcasted_iota(jnp.int32, (1, S), 1) >= sl

    zrow = jnp.zeros((1, E), jnp.float32)
    xcat = jnp.concatenate(
        [jnp.concatenate([zrow, x[:-1]], axis=0), x,
         jnp.concatenate([x[1:], zrow], axis=0)], axis=1)
    xcat_b = xcat.astype(jnp.bfloat16)

    def mmf32(a, w):
        return lax.dot_general(a, w, (((1,), (0,)), ((), ())),
                               preferred_element_type=jnp.float32)

    def rowdot(a, h):
        return lax.dot_general(a, h, (((1,), (1,)), ((), ())),
                               preferred_element_type=jnp.float32)

    ones_row = jnp.full((1, FILT), 1.0 / FILT, jnp.float32)

    h_all = jnp.maximum(mmf32(xcat_b, w1_ref[...]) + b1_ref[...], 0.0)

    def predictor(i, out_ref):
        h = h_all[:, i * FILT:(i + 1) * FILT]
        m = jnp.mean(h, axis=-1, keepdims=True)
        q = jnp.mean(h * h, axis=-1, keepdims=True)
        s = lax.rsqrt(q - m * m + 1e-5)
        v = ((h - m) * s + beg1_ref[i][None, :]).astype(jnp.bfloat16)
        zr = jnp.zeros((1, FILT), jnp.bfloat16)
        vcat = jnp.concatenate(
            [jnp.concatenate([zr, v[:-1]], axis=0), v,
             jnp.concatenate([v[1:], zr], axis=0)], axis=1)
        h2 = jnp.maximum(mmf32(vcat, w2_ref[i]) + b2_ref[i][None, :], 0.0)
        m2 = rowdot(ones_row, h2)
        q2 = rowdot(ones_row, h2 * h2)
        s2 = lax.rsqrt(q2 - m2 * m2 + 1e-5)
        u = rowdot(wlp_ref[i], h2)
        o = s2 * (u - m2 * gwsum_ref[i, 0]) + blp_ref[i, 0]
        out_ref[pl.ds(r, 1), :] = jnp.where(padr, 0.0, o)

    predictor(0, logd_ref)
    predictor(1, pp_ref)
    predictor(2, ep_ref)


def _b_call(src_lens, blp, gwsum, x, w1all, b1all, beg1, W2p, b2s, wlp):
    out_shape = (
        jax.ShapeDtypeStruct((B, S), jnp.float32),
        jax.ShapeDtypeStruct((B, S), jnp.float32),
        jax.ShapeDtypeStruct((B, S), jnp.float32),
    )

    def full(shape):
        return pl.BlockSpec(shape, lambda b, n=len(shape): (0,) * n)

    def row8(n):
        return pl.BlockSpec((8, n), lambda b: (b // 8, 0))

    return pl.pallas_call(
        _b_body,
        grid=(B,),
        in_specs=[
            pl.BlockSpec(memory_space=pltpu.SMEM),
            pl.BlockSpec(memory_space=pltpu.SMEM),
            pl.BlockSpec(memory_space=pltpu.SMEM),
            pl.BlockSpec((1, S, E), lambda b: (b, 0, 0)),
            full((3 * E, 3 * FILT)),
            full((1, 3 * FILT)),
            full((3, FILT)),
            full((3, 3 * FILT, FILT)),
            full((3, FILT)),
            full((3, 1, FILT)),
        ],
        out_specs=[row8(S), row8(S), row8(S)],
        out_shape=out_shape,
    )(src_lens, blp, gwsum, x, w1all, b1all, beg1, W2p, b2s, wlp)


def _sc_gather(x2f, gidxf):
    mesh = plsc.VectorSubcoreMesh(core_axis_name="c", subcore_axis_name="s")

    @functools.partial(
        pl.kernel,
        mesh=mesh,
        out_type=jax.ShapeDtypeStruct((B * MAXL, E), jnp.float32),
        scratch_types=[
            pltpu.VMEM((FPW,), jnp.int32),
            pltpu.VMEM((CH, E), jnp.float32),
            pltpu.VMEM((CH, E), jnp.float32),
            pltpu.SemaphoreType.DMA,
            pltpu.SemaphoreType.DMA,
        ],
    )
    def k(x2_hbm, gidx_hbm, out_hbm, idx_v, buf0, buf1, sem0, sem1):
        cid = lax.axis_index("c")
        sid = lax.axis_index("s")
        wid = sid * NC + cid
        base = wid * FPW
        pltpu.sync_copy(gidx_hbm.at[pl.ds(base, FPW)], idx_v)
        bufs = (buf0, buf1)
        sems = (sem0, sem1)
        nch = FPW // CH
        cps = [None] * nch
        cps[0] = pltpu.async_copy(x2_hbm.at[idx_v.at[pl.ds(0, CH)]],
                                  bufs[0], sems[0])
        for ci in range(nch):
            if ci + 1 < nch:
                cps[ci + 1] = pltpu.async_copy(
                    x2_hbm.at[idx_v.at[pl.ds((ci + 1) * CH, CH)]],
                    bufs[(ci + 1) % 2], sems[(ci + 1) % 2])
            cps[ci].wait()
            pltpu.sync_copy(bufs[ci % 2],
                            out_hbm.at[pl.ds(base + ci * CH, CH)])

    return k(x2f, gidxf)


def kernel(x, src_lens, duration_target, pitch_target, energy_target,
           max_len, W1s, b1s, g1s, be1s, W2s, b2s, g2s, be2s, Wls, bls,
           pitch_emb, energy_emb, pitch_bins, energy_bins):
    ninf = jnp.full((1,), -jnp.inf, jnp.float32)
    pinf = jnp.full((1,), jnp.inf, jnp.float32)
    blo_p = jnp.concatenate([ninf, pitch_bins]).reshape(N_BINS, 1)
    bhi_p = jnp.concatenate([pitch_bins, pinf]).reshape(N_BINS, 1)
    blo_e = jnp.concatenate([ninf, energy_bins]).reshape(N_BINS, 1)
    bhi_e = jnp.concatenate([energy_bins, pinf]).reshape(N_BINS, 1)

    x2p, gidx2, mlen3, mask2 = _a_call(
        src_lens, x, duration_target, pitch_target, energy_target,
        pitch_emb, energy_emb, blo_p, bhi_p, blo_e, bhi_e)

    mel = _sc_gather(x2p.reshape((B + 1) * S, E), gidx2.reshape(B * MAXL))

    w1all = jnp.concatenate(
        [W1s[i].reshape(3 * E, FILT) for i in range(3)],
        axis=1).astype(jnp.bfloat16)
    b1all = b1s.reshape(1, 3 * FILT)
    beg1 = be1s / g1s
    W2p = (W2s * g1s[:, None, :, None]).reshape(
        3, 3 * FILT, FILT).astype(jnp.bfloat16)
    wlp = (g2s * Wls[:, :, 0]).reshape(3, 1, FILT)
    blp = bls + jnp.sum(be2s * Wls[:, :, 0], axis=1, keepdims=True)
    gwsum = jnp.sum(wlp[:, 0, :], axis=1, keepdims=True)

    logd, pp, ep = _b_call(src_lens, blp, gwsum, x, w1all, b1all, beg1,
                           W2p, b2s, wlp)

    return (mel.reshape(B, MAXL, E), logd, pp, ep,
            mlen3[:, 0, 0], mask2.astype(bool))


# speedup vs baseline: 1.0958x; 1.0100x over previous
import functools

import jax
import jax.numpy as jnp
from jax import lax
from jax.experimental import pallas as pl
from jax.experimental.pallas import tpu as pltpu
from jax.experimental.pallas import tpu_sc as plsc

B, S, E = 16, 512, 256
FILT = 256
N_BINS = 256
MAXL = 2048
NC, NS = 2, 16
NW = NC * NS
FPW = (B * MAXL) // NW
CH = 128


def _a_body(sl_ref, x_ref, d_ref, pt_ref, et_ref, pemb_ref, eemb_ref,
            blo_p_ref, bhi_p_ref, blo_e_ref, bhi_e_ref,
            x2_ref, gidx_ref, mlen_ref, mask_ref):
    b = pl.program_id(0)

    @pl.when(b == B)
    def _zero_block():
        x2_ref[...] = jnp.zeros((1, S, E), jnp.float32)

    @pl.when(b < B)
    def _main():
        r = lax.rem(b, 8)
        x = x_ref[0]
        sl = sl_ref[b]

        pt = pt_ref[pl.ds(r, 1), :]
        ohT_p = ((pt >= blo_p_ref[...]).astype(jnp.float32)
                 - (pt >= bhi_p_ref[...]).astype(jnp.float32))
        et = et_ref[pl.ds(r, 1), :]
        ohT_e = ((et >= blo_e_ref[...]).astype(jnp.float32)
                 - (et >= bhi_e_ref[...]).astype(jnp.float32))

        def dotT(ohT, emb):
            return lax.dot_general(ohT, emb, (((0,), (0,)), ((), ())),
                                   preferred_element_type=jnp.float32)

        x2_ref[0] = x + dotT(ohT_p, pemb_ref[...]) + dotT(ohT_e, eemb_ref[...])

        drow = d_ref[pl.ds(r, 1), :].astype(jnp.float32)
        tokr = lax.broadcasted_iota(jnp.int32, (1, S), 1)
        dmask = jnp.where(tokr >= sl, 0.0, drow)
        ii = lax.broadcasted_iota(jnp.int32, (S, S), 0)
        jj = lax.broadcasted_iota(jnp.int32, (S, S), 1)
        ltri = (jj <= ii).astype(jnp.float32)
        cum_col = lax.dot_general(ltri, dmask, (((1,), (1,)), ((), ())),
                                  preferred_element_type=jnp.float32)

        total = jnp.sum(dmask).astype(jnp.int32)
        mlen = jnp.minimum(total, MAXL)
        mlen_ref[0] = jnp.full((1, 128), mlen, jnp.int32)

        frames = lax.broadcasted_iota(jnp.int32, (1, MAXL), 1)
        gefT = (cum_col <= frames.astype(jnp.float32)).astype(jnp.float32)
        ones = jnp.full((1, S), 1, jnp.float32)
        idxf = lax.dot_general(ones, gefT, (((1,), (0,)), ((), ())),
                               preferred_element_type=jnp.float32)
        idx = jnp.clip(idxf.astype(jnp.int32), 0, S - 1)
        zidx = B * S + (frames & (S - 1))
        gidx_ref[pl.ds(r, 1), :] = jnp.where(frames < mlen, idx + b * S, zidx)
        mask_ref[pl.ds(r, 1), :] = (frames >= mlen).astype(jnp.int32)


def _a_call(src_lens, x, dur, pt, et, pemb, eemb,
            blo_p, bhi_p, blo_e, bhi_e):
    out_shape = (
        jax.ShapeDtypeStruct((B + 1, S, E), jnp.float32),
        jax.ShapeDtypeStruct((B, MAXL), jnp.int32),
        jax.ShapeDtypeStruct((B, 1, 128), jnp.int32),
        jax.ShapeDtypeStruct((B, MAXL), jnp.int32),
    )

    def full(shape):
        return pl.BlockSpec(shape, lambda b, n=len(shape): (0,) * n)

    def row8(n):
        return pl.BlockSpec((8, n), lambda b: (jnp.minimum(b, B - 1) // 8, 0))

    return pl.pallas_call(
        _a_body,
        grid=(B + 1,),
        in_specs=[
            pl.BlockSpec(memory_space=pltpu.SMEM),
            pl.BlockSpec((1, S, E),
                         lambda b: (jnp.minimum(b, B - 1), 0, 0)),
            row8(S),
            row8(S),
            row8(S),
            full((N_BINS, E)), full((N_BINS, E)),
            full((N_BINS, 1)), full((N_BINS, 1)),
            full((N_BINS, 1)), full((N_BINS, 1)),
        ],
        out_specs=[
            pl.BlockSpec((1, S, E), lambda b: (b, 0, 0)),
            row8(MAXL),
            pl.BlockSpec((1, 1, 128),
                         lambda b: (jnp.minimum(b, B - 1), 0, 0)),
            row8(MAXL),
        ],
        out_shape=out_shape,
    )(src_lens, x, dur, pt, et, pemb, eemb, blo_p, bhi_p, blo_e, bhi_e)


def _b_body(sl_ref, blp_ref, x_ref, w1_ref, b1_ref, beg1_ref,
            w2_ref, b2_ref, wlp_ref,
            logd_ref, pp_ref, ep_ref):
    b = pl.program_id(0)
    r = lax.rem(b, 8)
    x = x_ref[0]
    sl = sl_ref[b]
    padr = lax.broadcasted_iota(jnp.int32, (1, S), 1) >= sl

    zrow = jnp.zeros((1, E), jnp.float32)
    xcat = jnp.concatenate(
        [jnp.concatenate([zrow, x[:-1]], axis=0), x,
         jnp.concatenate([x[1:], zrow], axis=0)], axis=1)
    xcat_b = xcat.astype(jnp.bfloat16)

    def mmf32(a, w):
        return lax.dot_general(a, w, (((1,), (0,)), ((), ())),
                               preferred_element_type=jnp.float32)

    def rowdot(a, h):
        return lax.dot_general(a, h, (((1,), (1,)), ((), ())),
                               preferred_element_type=jnp.float32)

    h_all = jnp.maximum(mmf32(xcat_b, w1_ref[...]) + b1_ref[...], 0.0)

    def predictor(i, out_ref):
        h = h_all[:, i * FILT:(i + 1) * FILT]
        m = jnp.mean(h, axis=-1, keepdims=True)
        q = jnp.mean(h * h, axis=-1, keepdims=True)
        s = lax.rsqrt(q - m * m + 1e-5)
        v = ((h - m) * s + beg1_ref[i][None, :]).astype(jnp.bfloat16)
        zr = jnp.zeros((1, FILT), jnp.bfloat16)
        vcat = jnp.concatenate(
            [jnp.concatenate([zr, v[:-1]], axis=0), v,
             jnp.concatenate([v[1:], zr], axis=0)], axis=1)
        h2 = jnp.maximum(mmf32(vcat, w2_ref[i]) + b2_ref[i][None, :], 0.0)
        m2 = jnp.mean(h2, axis=-1, keepdims=True)
        q2 = jnp.mean(h2 * h2, axis=-1, keepdims=True)
        s2 = lax.rsqrt(q2 - m2 * m2 + 1e-5)
        c2 = (h2 - m2) * s2
        o = rowdot(wlp_ref[i], c2) + blp_ref[i, 0]
        out_ref[pl.ds(r, 1), :] = jnp.where(padr, 0.0, o)

    predictor(0, logd_ref)
    predictor(1, pp_ref)
    predictor(2, ep_ref)


def _b_call(src_lens, blp, x, w1all, b1all, beg1, W2p, b2s, wlp):
    out_shape = (
        jax.ShapeDtypeStruct((B, S), jnp.float32),
        jax.ShapeDtypeStruct((B, S), jnp.float32),
        jax.ShapeDtypeStruct((B, S), jnp.float32),
    )

    def full(shape):
        return pl.BlockSpec(shape, lambda b, n=len(shape): (0,) * n)

    def row8(n):
        return pl.BlockSpec((8, n), lambda b: (b // 8, 0))

    return pl.pallas_call(
        _b_body,
        grid=(B,),
        in_specs=[
            pl.BlockSpec(memory_space=pltpu.SMEM),
            pl.BlockSpec(memory_space=pltpu.SMEM),
            pl.BlockSpec((1, S, E), lambda b: (b, 0, 0)),
            full((3 * E, 3 * FILT)),
            full((1, 3 * FILT)),
            full((3, FILT)),
            full((3, 3 * FILT, FILT)),
            full((3, FILT)),
            full((3, 1, FILT)),
        ],
        out_specs=[row8(S), row8(S), row8(S)],
        out_shape=out_shape,
    )(src_lens, blp, x, w1all, b1all, beg1, W2p, b2s, wlp)


def _sc_gather(x2f, gidxf):
    mesh = plsc.VectorSubcoreMesh(core_axis_name="c", subcore_axis_name="s")

    @functools.partial(
        pl.kernel,
        mesh=mesh,
        out_type=jax.ShapeDtypeStruct((B * MAXL, E), jnp.float32),
        scratch_types=[
            pltpu.VMEM((FPW,), jnp.int32),
            pltpu.VMEM((CH, E), jnp.float32),
            pltpu.VMEM((CH, E), jnp.float32),
            pltpu.SemaphoreType.DMA,
            pltpu.SemaphoreType.DMA,
        ],
    )
    def k(x2_hbm, gidx_hbm, out_hbm, idx_v, buf0, buf1, sem0, sem1):
        cid = lax.axis_index("c")
        sid = lax.axis_index("s")
        wid = sid * NC + cid
        base = wid * FPW
        pltpu.sync_copy(gidx_hbm.at[pl.ds(base, FPW)], idx_v)
        bufs = (buf0, buf1)
        sems = (sem0, sem1)
        nch = FPW // CH
        cps = [None] * nch
        cps[0] = pltpu.async_copy(x2_hbm.at[idx_v.at[pl.ds(0, CH)]],
                                  bufs[0], sems[0])
        for ci in range(nch):
            if ci + 1 < nch:
                cps[ci + 1] = pltpu.async_copy(
                    x2_hbm.at[idx_v.at[pl.ds((ci + 1) * CH, CH)]],
                    bufs[(ci + 1) % 2], sems[(ci + 1) % 2])
            cps[ci].wait()
            pltpu.sync_copy(bufs[ci % 2],
                            out_hbm.at[pl.ds(base + ci * CH, CH)])

    return k(x2f, gidxf)


def kernel(x, src_lens, duration_target, pitch_target, energy_target,
           max_len, W1s, b1s, g1s, be1s, W2s, b2s, g2s, be2s, Wls, bls,
           pitch_emb, energy_emb, pitch_bins, energy_bins):
    ninf = jnp.full((1,), -jnp.inf, jnp.float32)
    pinf = jnp.full((1,), jnp.inf, jnp.float32)
    blo_p = jnp.concatenate([ninf, pitch_bins]).reshape(N_BINS, 1)
    bhi_p = jnp.concatenate([pitch_bins, pinf]).reshape(N_BINS, 1)
    blo_e = jnp.concatenate([ninf, energy_bins]).reshape(N_BINS, 1)
    bhi_e = jnp.concatenate([energy_bins, pinf]).reshape(N_BINS, 1)

    x2p, gidx2, mlen3, mask2 = _a_call(
        src_lens, x, duration_target, pitch_target, energy_target,
        pitch_emb, energy_emb, blo_p, bhi_p, blo_e, bhi_e)

    mel = _sc_gather(x2p.reshape((B + 1) * S, E), gidx2.reshape(B * MAXL))

    w1all = jnp.concatenate(
        [W1s[i].reshape(3 * E, FILT) for i in range(3)],
        axis=1).astype(jnp.bfloat16)
    b1all = b1s.reshape(1, 3 * FILT)
    beg1 = be1s / g1s
    W2p = (W2s * g1s[:, None, :, None]).reshape(
        3, 3 * FILT, FILT).astype(jnp.bfloat16)
    wlp = (g2s * Wls[:, :, 0]).reshape(3, 1, FILT)
    blp = bls + jnp.sum(be2s * Wls[:, :, 0], axis=1, keepdims=True)

    logd, pp, ep = _b_call(src_lens, blp, x, w1all, b1all, beg1,
                           W2p, b2s, wlp)

    return (mel.reshape(B, MAXL, E), logd, pp, ep,
            mlen3[:, 0, 0], mask2.astype(bool))


# speedup vs baseline: 1.1738x; 1.0711x over previous
import functools

import jax
import jax.numpy as jnp
from jax import lax
from jax.experimental import pallas as pl
from jax.experimental.pallas import tpu as pltpu
from jax.experimental.pallas import tpu_sc as plsc

B, S, E = 16, 512, 256
FILT = 256
N_BINS = 256
MAXL = 2048
NC, NS = 2, 16
NW = NC * NS
FPW = (B * MAXL) // NW
CH = 128


def _a_body(sl_ref, x_ref, d_ref, pt_ref, et_ref, pemb_ref, eemb_ref,
            blo_p_ref, bhi_p_ref, blo_e_ref, bhi_e_ref,
            x2_ref, gidx_ref, mlen_ref, mask_ref):
    b = pl.program_id(0)

    @pl.when(b == B)
    def _zero_block():
        x2_ref[...] = jnp.zeros((1, S, E), jnp.float32)

    @pl.when(b < B)
    def _main():
        r = lax.rem(b, 8)
        x = x_ref[0]
        sl = sl_ref[b]

        pt = pt_ref[pl.ds(r, 1), :]
        ohT_p = ((pt >= blo_p_ref[...]).astype(jnp.float32)
                 - (pt >= bhi_p_ref[...]).astype(jnp.float32))
        et = et_ref[pl.ds(r, 1), :]
        ohT_e = ((et >= blo_e_ref[...]).astype(jnp.float32)
                 - (et >= bhi_e_ref[...]).astype(jnp.float32))

        def dotT(ohT, emb):
            return lax.dot_general(ohT, emb, (((0,), (0,)), ((), ())),
                                   preferred_element_type=jnp.float32)

        x2_ref[0] = x + dotT(ohT_p, pemb_ref[...]) + dotT(ohT_e, eemb_ref[...])

        drow = d_ref[pl.ds(r, 1), :].astype(jnp.float32)
        tokr = lax.broadcasted_iota(jnp.int32, (1, S), 1)
        dmask = jnp.where(tokr >= sl, 0.0, drow)
        ii = lax.broadcasted_iota(jnp.int32, (S, S), 0)
        jj = lax.broadcasted_iota(jnp.int32, (S, S), 1)
        ltri = (jj <= ii).astype(jnp.float32)
        cum_col = lax.dot_general(ltri, dmask, (((1,), (1,)), ((), ())),
                                  preferred_element_type=jnp.float32)

        total = jnp.sum(dmask).astype(jnp.int32)
        mlen = jnp.minimum(total, MAXL)
        mlen_ref[0] = jnp.full((1, 128), mlen, jnp.int32)

        frames = lax.broadcasted_iota(jnp.int32, (1, MAXL), 1)
        gefT = (cum_col <= frames.astype(jnp.float32)).astype(jnp.float32)
        ones = jnp.full((1, S), 1, jnp.float32)
        idxf = lax.dot_general(ones, gefT, (((1,), (0,)), ((), ())),
                               preferred_element_type=jnp.float32)
        idx = jnp.clip(idxf.astype(jnp.int32), 0, S - 1)
        zidx = B * S + (frames & (S - 1))
        gidx_ref[pl.ds(r, 1), :] = jnp.where(frames < mlen, idx + b * S, zidx)
        mask_ref[pl.ds(r, 1), :] = (frames >= mlen).astype(jnp.int32)


def _a_call(src_lens, x, dur, pt, et, pemb, eemb,
            blo_p, bhi_p, blo_e, bhi_e):
    out_shape = (
        jax.ShapeDtypeStruct((B + 1, S, E), jnp.float32),
        jax.ShapeDtypeStruct((B, MAXL), jnp.int32),
        jax.ShapeDtypeStruct((B, 1, 128), jnp.int32),
        jax.ShapeDtypeStruct((B, MAXL), jnp.int32),
    )

    def full(shape):
        return pl.BlockSpec(shape, lambda b, n=len(shape): (0,) * n)

    def row8(n):
        return pl.BlockSpec((8, n), lambda b: (jnp.minimum(b, B - 1) // 8, 0))

    return pl.pallas_call(
        _a_body,
        grid=(B + 1,),
        in_specs=[
            pl.BlockSpec(memory_space=pltpu.SMEM),
            pl.BlockSpec((1, S, E),
                         lambda b: (jnp.minimum(b, B - 1), 0, 0)),
            row8(S),
            row8(S),
            row8(S),
            full((N_BINS, E)), full((N_BINS, E)),
            full((N_BINS, 1)), full((N_BINS, 1)),
            full((N_BINS, 1)), full((N_BINS, 1)),
        ],
        out_specs=[
            pl.BlockSpec((1, S, E), lambda b: (b, 0, 0)),
            row8(MAXL),
            pl.BlockSpec((1, 1, 128),
                         lambda b: (jnp.minimum(b, B - 1), 0, 0)),
            row8(MAXL),
        ],
        out_shape=out_shape,
    )(src_lens, x, dur, pt, et, pemb, eemb, blo_p, bhi_p, blo_e, bhi_e)


def _b_body(sl_ref, blp_ref, x_ref, w1_ref, b1_ref, beg1_ref,
            w2_ref, b2_ref, wlp_ref,
            logd_ref, pp_ref, ep_ref):
    b = pl.program_id(0)

    def mmf32(a, w):
        return lax.dot_general(a, w, (((1,), (0,)), ((), ())),
                               preferred_element_type=jnp.float32)

    def rowdot(a, h):
        return lax.dot_general(a, h, (((1,), (1,)), ((), ())),
                               preferred_element_type=jnp.float32)

    def mkcat(v, zr):
        return jnp.concatenate(
            [jnp.concatenate([zr, v[:-1]], axis=0), v,
             jnp.concatenate([v[1:], zr], axis=0)], axis=1)

    zrow = jnp.zeros((1, E), jnp.float32)
    xcat2 = jnp.concatenate(
        [mkcat(x_ref[0], zrow), mkcat(x_ref[1], zrow)],
        axis=0).astype(jnp.bfloat16)

    h_all = jnp.maximum(mmf32(xcat2, w1_ref[...]) + b1_ref[...], 0.0)

    def predictor(i, out_ref):
        h = h_all[:, i * FILT:(i + 1) * FILT]
        m = jnp.mean(h, axis=-1, keepdims=True)
        q = jnp.mean(h * h, axis=-1, keepdims=True)
        s = lax.rsqrt(q - m * m + 1e-5)
        v = ((h - m) * s + beg1_ref[i][None, :]).astype(jnp.bfloat16)
        zr = jnp.zeros((1, FILT), jnp.bfloat16)
        vcat2 = jnp.concatenate(
            [mkcat(v[:S], zr), mkcat(v[S:], zr)], axis=0)
        h2 = jnp.maximum(mmf32(vcat2, w2_ref[i]) + b2_ref[i][None, :], 0.0)
        m2 = jnp.mean(h2, axis=-1, keepdims=True)
        q2 = jnp.mean(h2 * h2, axis=-1, keepdims=True)
        s2 = lax.rsqrt(q2 - m2 * m2 + 1e-5)
        c2 = (h2 - m2) * s2
        o = rowdot(wlp_ref[i], c2) + blp_ref[i, 0]
        for k in range(2):
            sl = sl_ref[2 * b + k]
            padr = lax.broadcasted_iota(jnp.int32, (1, S), 1) >= sl
            out_ref[pl.ds(2 * lax.rem(b, 4) + k, 1), :] = jnp.where(
                padr, 0.0, o[:, k * S:(k + 1) * S])

    predictor(0, logd_ref)
    predictor(1, pp_ref)
    predictor(2, ep_ref)


def _b_call(src_lens, blp, x, w1all, b1all, beg1, W2p, b2s, wlp):
    out_shape = (
        jax.ShapeDtypeStruct((B, S), jnp.float32),
        jax.ShapeDtypeStruct((B, S), jnp.float32),
        jax.ShapeDtypeStruct((B, S), jnp.float32),
    )

    def full(shape):
        return pl.BlockSpec(shape, lambda b, n=len(shape): (0,) * n)

    def row8(n):
        return pl.BlockSpec((8, n), lambda b: (b // 4, 0))

    return pl.pallas_call(
        _b_body,
        grid=(B // 2,),
        in_specs=[
            pl.BlockSpec(memory_space=pltpu.SMEM),
            pl.BlockSpec(memory_space=pltpu.SMEM),
            pl.BlockSpec((2, S, E), lambda b: (b, 0, 0)),
            full((3 * E, 3 * FILT)),
            full((1, 3 * FILT)),
            full((3, FILT)),
            full((3, 3 * FILT, FILT)),
            full((3, FILT)),
            full((3, 1, FILT)),
        ],
        out_specs=[row8(S), row8(S), row8(S)],
        out_shape=out_shape,
    )(src_lens, blp, x, w1all, b1all, beg1, W2p, b2s, wlp)


def _sc_gather(x2f, gidxf):
    mesh = plsc.VectorSubcoreMesh(core_axis_name="c", subcore_axis_name="s")

    @functools.partial(
        pl.kernel,
        mesh=mesh,
        out_type=jax.ShapeDtypeStruct((B * MAXL, E), jnp.float32),
        scratch_types=[
            pltpu.VMEM((FPW,), jnp.int32),
            pltpu.VMEM((CH, E), jnp.float32),
            pltpu.VMEM((CH, E), jnp.float32),
            pltpu.SemaphoreType.DMA,
            pltpu.SemaphoreType.DMA,
        ],
    )
    def k(x2_hbm, gidx_hbm, out_hbm, idx_v, buf0, buf1, sem0, sem1):
        cid = lax.axis_index("c")
        sid = lax.axis_index("s")
        wid = sid * NC + cid
        base = wid * FPW
        pltpu.sync_copy(gidx_hbm.at[pl.ds(base, FPW)], idx_v)
        bufs = (buf0, buf1)
        sems = (sem0, sem1)
        nch = FPW // CH
        cps = [None] * nch
        cps[0] = pltpu.async_copy(x2_hbm.at[idx_v.at[pl.ds(0, CH)]],
                                  bufs[0], sems[0])
        for ci in range(nch):
            if ci + 1 < nch:
                cps[ci + 1] = pltpu.async_copy(
                    x2_hbm.at[idx_v.at[pl.ds((ci + 1) * CH, CH)]],
                    bufs[(ci + 1) % 2], sems[(ci + 1) % 2])
            cps[ci].wait()
            pltpu.sync_copy(bufs[ci % 2],
                            out_hbm.at[pl.ds(base + ci * CH, CH)])

    return k(x2f, gidxf)


def kernel(x, src_lens, duration_target, pitch_target, energy_target,
           max_len, W1s, b1s, g1s, be1s, W2s, b2s, g2s, be2s, Wls, bls,
           pitch_emb, energy_emb, pitch_bins, energy_bins):
    ninf = jnp.full((1,), -jnp.inf, jnp.float32)
    pinf = jnp.full((1,), jnp.inf, jnp.float32)
    blo_p = jnp.concatenate([ninf, pitch_bins]).reshape(N_BINS, 1)
    bhi_p = jnp.concatenate([pitch_bins, pinf]).reshape(N_BINS, 1)
    blo_e = jnp.concatenate([ninf, energy_bins]).reshape(N_BINS, 1)
    bhi_e = jnp.concatenate([energy_bins, pinf]).reshape(N_BINS, 1)

    x2p, gidx2, mlen3, mask2 = _a_call(
        src_lens, x, duration_target, pitch_target, energy_target,
        pitch_emb, energy_emb, blo_p, bhi_p, blo_e, bhi_e)

    mel = _sc_gather(x2p.reshape((B + 1) * S, E), gidx2.reshape(B * MAXL))

    w1all = jnp.concatenate(
        [W1s[i].reshape(3 * E, FILT) for i in range(3)],
        axis=1).astype(jnp.bfloat16)
    b1all = b1s.reshape(1, 3 * FILT)
    beg1 = be1s / g1s
    W2p = (W2s * g1s[:, None, :, None]).reshape(
        3, 3 * FILT, FILT).astype(jnp.bfloat16)
    wlp = (g2s * Wls[:, :, 0]).reshape(3, 1, FILT)
    blp = bls + jnp.sum(be2s * Wls[:, :, 0], axis=1, keepdims=True)

    logd, pp, ep = _b_call(src_lens, blp, x, w1all, b1all, beg1,
                           W2p, b2s, wlp)

    return (mel.reshape(B, MAXL, E), logd, pp, ep,
            mlen3[:, 0, 0], mask2.astype(bool))


# speedup vs baseline: 1.1753x; 1.0014x over previous
import functools

import jax
import jax.numpy as jnp
from jax import lax
from jax.experimental import pallas as pl
from jax.experimental.pallas import tpu as pltpu
from jax.experimental.pallas import tpu_sc as plsc

B, S, E = 16, 512, 256
FILT = 256
N_BINS = 256
MAXL = 2048
NC, NS = 2, 16
NW = NC * NS
FPW = (B * MAXL) // NW
CH = 128


def _a_body(sl_ref, x_ref, d_ref, pt_ref, et_ref, pemb_ref, eemb_ref,
            blo_p_ref, bhi_p_ref, blo_e_ref, bhi_e_ref,
            x2_ref, gidx_ref, mlen_ref, mask_ref):
    b = pl.program_id(0)

    @pl.when(b == B)
    def _zero_block():
        x2_ref[...] = jnp.zeros((1, S, E), jnp.float32)

    @pl.when(b < B)
    def _main():
        r = lax.rem(b, 8)
        x = x_ref[0]
        sl = sl_ref[b]

        pt = pt_ref[pl.ds(r, 1), :]
        ohT_p = ((pt >= blo_p_ref[...]).astype(jnp.float32)
                 - (pt >= bhi_p_ref[...]).astype(jnp.float32))
        et = et_ref[pl.ds(r, 1), :]
        ohT_e = ((et >= blo_e_ref[...]).astype(jnp.float32)
                 - (et >= bhi_e_ref[...]).astype(jnp.float32))

        def dotT(ohT, emb):
            return lax.dot_general(ohT, emb, (((0,), (0,)), ((), ())),
                                   preferred_element_type=jnp.float32)

        x2_ref[0] = x + dotT(ohT_p, pemb_ref[...]) + dotT(ohT_e, eemb_ref[...])

        drow = d_ref[pl.ds(r, 1), :].astype(jnp.float32)
        tokr = lax.broadcasted_iota(jnp.int32, (1, S), 1)
        dmask = jnp.where(tokr >= sl, 0.0, drow)
        ii = lax.broadcasted_iota(jnp.int32, (S, S), 0)
        jj = lax.broadcasted_iota(jnp.int32, (S, S), 1)
        ltri = (jj <= ii).astype(jnp.float32)
        cum_col = lax.dot_general(ltri, dmask, (((1,), (1,)), ((), ())),
                                  preferred_element_type=jnp.float32)

        total = jnp.sum(dmask).astype(jnp.int32)
        mlen = jnp.minimum(total, MAXL)
        mlen_ref[0] = jnp.full((1, 128), mlen, jnp.int32)

        frames = lax.broadcasted_iota(jnp.int32, (1, MAXL), 1)
        gefT = (cum_col <= frames.astype(jnp.float32)).astype(jnp.float32)
        ones = jnp.full((1, S), 1, jnp.float32)
        idxf = lax.dot_general(ones, gefT, (((1,), (0,)), ((), ())),
                               preferred_element_type=jnp.float32)
        idx = jnp.clip(idxf.astype(jnp.int32), 0, S - 1)
        zidx = B * S + (frames & (S - 1))
        gidx_ref[pl.ds(r, 1), :] = jnp.where(frames < mlen, idx + b * S, zidx)
        mask_ref[pl.ds(r, 1), :] = (frames >= mlen).astype(jnp.int32)


def _a_call(src_lens, x, dur, pt, et, pemb, eemb,
            blo_p, bhi_p, blo_e, bhi_e):
    out_shape = (
        jax.ShapeDtypeStruct((B + 1, S, E), jnp.float32),
        jax.ShapeDtypeStruct((B, MAXL), jnp.int32),
        jax.ShapeDtypeStruct((B, 1, 128), jnp.int32),
        jax.ShapeDtypeStruct((B, MAXL), jnp.int32),
    )

    def full(shape):
        return pl.BlockSpec(shape, lambda b, n=len(shape): (0,) * n)

    def row8(n):
        return pl.BlockSpec((8, n), lambda b: (jnp.minimum(b, B - 1) // 8, 0))

    return pl.pallas_call(
        _a_body,
        grid=(B + 1,),
        in_specs=[
            pl.BlockSpec(memory_space=pltpu.SMEM),
            pl.BlockSpec((1, S, E),
                         lambda b: (jnp.minimum(b, B - 1), 0, 0)),
            row8(S),
            row8(S),
            row8(S),
            full((N_BINS, E)), full((N_BINS, E)),
            full((N_BINS, 1)), full((N_BINS, 1)),
            full((N_BINS, 1)), full((N_BINS, 1)),
        ],
        out_specs=[
            pl.BlockSpec((1, S, E), lambda b: (b, 0, 0)),
            row8(MAXL),
            pl.BlockSpec((1, 1, 128),
                         lambda b: (jnp.minimum(b, B - 1), 0, 0)),
            row8(MAXL),
        ],
        out_shape=out_shape,
    )(src_lens, x, dur, pt, et, pemb, eemb, blo_p, bhi_p, blo_e, bhi_e)


def _b_body(sl_ref, blp_ref, x_ref, w1_ref, b1_ref, beg1_ref,
            w2_ref, b2_ref, wlp_ref,
            logd_ref, pp_ref, ep_ref):
    b = pl.program_id(0)

    def mmf32(a, w):
        return lax.dot_general(a, w, (((1,), (0,)), ((), ())),
                               preferred_element_type=jnp.float32)

    def rowdot(a, h):
        return lax.dot_general(a, h, (((1,), (1,)), ((), ())),
                               preferred_element_type=jnp.float32)

    def mkcat(v, zr):
        return jnp.concatenate(
            [jnp.concatenate([zr, v[:-1]], axis=0), v,
             jnp.concatenate([v[1:], zr], axis=0)], axis=1)

    zrow = jnp.zeros((1, E), jnp.float32)
    xcat2 = jnp.concatenate(
        [mkcat(x_ref[k], zrow) for k in range(4)],
        axis=0).astype(jnp.bfloat16)

    h_all = jnp.maximum(mmf32(xcat2, w1_ref[...]) + b1_ref[...], 0.0)

    def predictor(i, out_ref):
        h = h_all[:, i * FILT:(i + 1) * FILT]
        m = jnp.mean(h, axis=-1, keepdims=True)
        q = jnp.mean(h * h, axis=-1, keepdims=True)
        s = lax.rsqrt(q - m * m + 1e-5)
        v = ((h - m) * s + beg1_ref[i][None, :]).astype(jnp.bfloat16)
        zr = jnp.zeros((1, FILT), jnp.bfloat16)
        vcat2 = jnp.concatenate(
            [mkcat(v[k * S:(k + 1) * S], zr) for k in range(4)], axis=0)
        h2 = jnp.maximum(mmf32(vcat2, w2_ref[i]) + b2_ref[i][None, :], 0.0)
        m2 = jnp.mean(h2, axis=-1, keepdims=True)
        q2 = jnp.mean(h2 * h2, axis=-1, keepdims=True)
        s2 = lax.rsqrt(q2 - m2 * m2 + 1e-5)
        c2 = (h2 - m2) * s2
        o = rowdot(wlp_ref[i], c2) + blp_ref[i, 0]
        for k in range(4):
            sl = sl_ref[4 * b + k]
            padr = lax.broadcasted_iota(jnp.int32, (1, S), 1) >= sl
            out_ref[pl.ds(4 * lax.rem(b, 2) + k, 1), :] = jnp.where(
                padr, 0.0, o[:, k * S:(k + 1) * S])

    predictor(0, logd_ref)
    predictor(1, pp_ref)
    predictor(2, ep_ref)


def _b_call(src_lens, blp, x, w1all, b1all, beg1, W2p, b2s, wlp):
    out_shape = (
        jax.ShapeDtypeStruct((B, S), jnp.float32),
        jax.ShapeDtypeStruct((B, S), jnp.float32),
        jax.ShapeDtypeStruct((B, S), jnp.float32),
    )

    def full(shape):
        return pl.BlockSpec(shape, lambda b, n=len(shape): (0,) * n)

    def row8(n):
        return pl.BlockSpec((8, n), lambda b: (b // 2, 0))

    return pl.pallas_call(
        _b_body,
        grid=(B // 4,),
        in_specs=[
            pl.BlockSpec(memory_space=pltpu.SMEM),
            pl.BlockSpec(memory_space=pltpu.SMEM),
            pl.BlockSpec((4, S, E), lambda b: (b, 0, 0)),
            full((3 * E, 3 * FILT)),
            full((1, 3 * FILT)),
            full((3, FILT)),
            full((3, 3 * FILT, FILT)),
            full((3, FILT)),
            full((3, 1, FILT)),
        ],
        out_specs=[row8(S), row8(S), row8(S)],
        out_shape=out_shape,
    )(src_lens, blp, x, w1all, b1all, beg1, W2p, b2s, wlp)


def _sc_gather(x2f, gidxf):
    mesh = plsc.VectorSubcoreMesh(core_axis_name="c", subcore_axis_name="s")

    @functools.partial(
        pl.kernel,
        mesh=mesh,
        out_type=jax.ShapeDtypeStruct((B * MAXL, E), jnp.float32),
        scratch_types=[
            pltpu.VMEM((FPW,), jnp.int32),
            pltpu.VMEM((CH, E), jnp.float32),
            pltpu.VMEM((CH, E), jnp.float32),
            pltpu.SemaphoreType.DMA,
            pltpu.SemaphoreType.DMA,
        ],
    )
    def k(x2_hbm, gidx_hbm, out_hbm, idx_v, buf0, buf1, sem0, sem1):
        cid = lax.axis_index("c")
        sid = lax.axis_index("s")
        wid = sid * NC + cid
        base = wid * FPW
        pltpu.sync_copy(gidx_hbm.at[pl.ds(base, FPW)], idx_v)
        bufs = (buf0, buf1)
        sems = (sem0, sem1)
        nch = FPW // CH
        cps = [None] * nch
        cps[0] = pltpu.async_copy(x2_hbm.at[idx_v.at[pl.ds(0, CH)]],
                                  bufs[0], sems[0])
        for ci in range(nch):
            if ci + 1 < nch:
                cps[ci + 1] = pltpu.async_copy(
                    x2_hbm.at[idx_v.at[pl.ds((ci + 1) * CH, CH)]],
                    bufs[(ci + 1) % 2], sems[(ci + 1) % 2])
            cps[ci].wait()
            pltpu.sync_copy(bufs[ci % 2],
                            out_hbm.at[pl.ds(base + ci * CH, CH)])

    return k(x2f, gidxf)


def kernel(x, src_lens, duration_target, pitch_target, energy_target,
           max_len, W1s, b1s, g1s, be1s, W2s, b2s, g2s, be2s, Wls, bls,
           pitch_emb, energy_emb, pitch_bins, energy_bins):
    ninf = jnp.full((1,), -jnp.inf, jnp.float32)
    pinf = jnp.full((1,), jnp.inf, jnp.float32)
    blo_p = jnp.concatenate([ninf, pitch_bins]).reshape(N_BINS, 1)
    bhi_p = jnp.concatenate([pitch_bins, pinf]).reshape(N_BINS, 1)
    blo_e = jnp.concatenate([ninf, energy_bins]).reshape(N_BINS, 1)
    bhi_e = jnp.concatenate([energy_bins, pinf]).reshape(N_BINS, 1)

    x2p, gidx2, mlen3, mask2 = _a_call(
        src_lens, x, duration_target, pitch_target, energy_target,
        pitch_emb, energy_emb, blo_p, bhi_p, blo_e, bhi_e)

    mel = _sc_gather(x2p.reshape((B + 1) * S, E), gidx2.reshape(B * MAXL))

    w1all = jnp.concatenate(
        [W1s[i].reshape(3 * E, FILT) for i in range(3)],
        axis=1).astype(jnp.bfloat16)
    b1all = b1s.reshape(1, 3 * FILT)
    beg1 = be1s / g1s
    W2p = (W2s * g1s[:, None, :, None]).reshape(
        3, 3 * FILT, FILT).astype(jnp.bfloat16)
    wlp = (g2s * Wls[:, :, 0]).reshape(3, 1, FILT)
    blp = bls + jnp.sum(be2s * Wls[:, :, 0], axis=1, keepdims=True)

    logd, pp, ep = _b_call(src_lens, blp, x, w1all, b1all, beg1,
                           W2p, b2s, wlp)

    return (mel.reshape(B, MAXL, E), logd, pp, ep,
            mlen3[:, 0, 0], mask2.astype(bool))
